# initial kernel scaffold (unmeasured)
import jax
import jax.numpy as jnp
from jax import lax
from jax.experimental import pallas as pl
from jax.experimental.pallas import tpu as pltpu

N_DEV = 32
M = 4096
N = 8192
CHUNK = M // N_DEV
N_STEPS = 2 * (N_DEV - 1)


def _ar_body(partial_ref, scale_ref, out_ref,
             buf, lbuf, send_sems, recv_sems, lsem, osem, credit):
    p = lax.axis_index("i")
    right = lax.rem(p + 1, N_DEV)
    left = lax.rem(p + N_DEV - 1, N_DEV)

    barrier = pltpu.get_barrier_semaphore()
    for nbr in (left, right):
        pl.semaphore_signal(barrier, inc=1, device_id=(nbr,),
                            device_id_type=pl.DeviceIdType.MESH)
    pl.semaphore_wait(barrier, 2)

    cp = pltpu.make_async_copy(
        partial_ref.at[pl.ds(p * CHUNK, CHUNK), :], buf.at[0], lsem.at[0])
    cp.start()
    cp.wait()

    scale = scale_ref[0]

    for s in range(N_STEPS):
        a, b = s % 2, (s + 1) % 2
        if s >= 1:
            pl.semaphore_wait(credit, 1)
        rdma = pltpu.make_async_remote_copy(
            src_ref=buf.at[a], dst_ref=buf.at[b],
            send_sem=send_sems.at[a], recv_sem=recv_sems.at[b],
            device_id=(right,), device_id_type=pl.DeviceIdType.MESH)
        rdma.start()

        c_recv = lax.rem(p - s - 1 + 2 * N_DEV, N_DEV)
        if s < N_DEV - 1:
            lcp = pltpu.make_async_copy(
                partial_ref.at[pl.ds(c_recv * CHUNK, CHUNK), :],
                lbuf.at[b], lsem.at[b])
            lcp.start()

        rdma.wait_send()
        if s < N_STEPS - 1:
            pl.semaphore_signal(credit, inc=1, device_id=(left,),
                                device_id_type=pl.DeviceIdType.MESH)
        rdma.wait_recv()

        if s < N_DEV - 2:
            lcp.wait()
            buf[b] = buf[b] + lbuf[b]
        elif s == N_DEV - 2:
            lcp.wait()
            v = jnp.maximum((buf[b] + lbuf[b]) * scale, 0.0)
            buf[b] = v
            ocp = pltpu.make_async_copy(
                buf.at[b], out_ref.at[pl.ds(c_recv * CHUNK, CHUNK), :], osem)
            ocp.start()
            ocp.wait()
        else:
            ocp = pltpu.make_async_copy(
                buf.at[b], out_ref.at[pl.ds(c_recv * CHUNK, CHUNK), :], osem)
            ocp.start()
            ocp.wait()


def kernel(x, w_mat, scale_x, scale_w):
    acc = lax.dot_general(x, w_mat, (((1,), (0,)), ((), ())),
                          preferred_element_type=jnp.int32)
    partial = acc.astype(jnp.float32)
    scale = (scale_x * scale_w).astype(jnp.float32)

    return pl.pallas_call(
        _ar_body,
        out_shape=jax.ShapeDtypeStruct((M, N), jnp.float32),
        in_specs=[
            pl.BlockSpec(memory_space=pltpu.ANY),
            pl.BlockSpec(memory_space=pltpu.SMEM),
        ],
        out_specs=pl.BlockSpec(memory_space=pltpu.ANY),
        scratch_shapes=[
            pltpu.VMEM((2, CHUNK, N), jnp.float32),
            pltpu.VMEM((2, CHUNK, N), jnp.float32),
            pltpu.SemaphoreType.DMA((2,)),
            pltpu.SemaphoreType.DMA((2,)),
            pltpu.SemaphoreType.DMA((2,)),
            pltpu.SemaphoreType.DMA,
            pltpu.SemaphoreType.REGULAR,
        ],
        compiler_params=pltpu.CompilerParams(collective_id=0),
    )(partial, scale)


# baseline (device time: 3199838 ns/iter reference)
import jax
import jax.numpy as jnp
from jax import lax
from jax.experimental import pallas as pl
from jax.experimental.pallas import tpu as pltpu

N_DEV = 32
M = 4096
N = 8192
CHUNK = M // N_DEV
N_STEPS = 2 * (N_DEV - 1)


def _ar_body(partial_ref, scale_ref, out_ref,
             buf, lbuf, send_sems, recv_sems, lsem, osem, credit):
    p = lax.axis_index("i")
    right = lax.rem(p + 1, N_DEV)
    left = lax.rem(p + N_DEV - 1, N_DEV)

    barrier = pltpu.get_barrier_semaphore()
    for nbr in (left, right):
        pl.semaphore_signal(barrier, inc=1, device_id=(nbr,),
                            device_id_type=pl.DeviceIdType.MESH)
    pl.semaphore_wait(barrier, 2)

    cp = pltpu.make_async_copy(
        partial_ref.at[pl.ds(p * CHUNK, CHUNK), :], buf.at[0], lsem.at[0])
    cp.start()
    cp.wait()

    scale = scale_ref[0]

    for s in range(N_STEPS):
        a, b = s % 2, (s + 1) % 2
        if s >= 1:
            pl.semaphore_wait(credit, 1)
        rdma = pltpu.make_async_remote_copy(
            src_ref=buf.at[a], dst_ref=buf.at[b],
            send_sem=send_sems.at[a], recv_sem=recv_sems.at[b],
            device_id=(right,), device_id_type=pl.DeviceIdType.MESH)
        rdma.start()

        c_recv = lax.rem(p - s - 1 + 2 * N_DEV, N_DEV)
        if s < N_DEV - 1:
            lcp = pltpu.make_async_copy(
                partial_ref.at[pl.ds(c_recv * CHUNK, CHUNK), :],
                lbuf.at[b], lsem.at[b])
            lcp.start()

        rdma.wait_send()
        if s < N_STEPS - 1:
            pl.semaphore_signal(credit, inc=1, device_id=(left,),
                                device_id_type=pl.DeviceIdType.MESH)
        rdma.wait_recv()

        if s < N_DEV - 2:
            lcp.wait()
            buf[b] = buf[b] + lbuf[b]
        elif s == N_DEV - 2:
            lcp.wait()
            v = jnp.maximum((buf[b] + lbuf[b]) * scale, 0.0)
            buf[b] = v
            ocp = pltpu.make_async_copy(
                buf.at[b], out_ref.at[pl.ds(c_recv * CHUNK, CHUNK), :], osem)
            ocp.start()
            ocp.wait()
        else:
            ocp = pltpu.make_async_copy(
                buf.at[b], out_ref.at[pl.ds(c_recv * CHUNK, CHUNK), :], osem)
            ocp.start()
            ocp.wait()


def kernel(x, w_mat, scale_x, scale_w):
    acc = lax.dot_general(x, w_mat, (((1,), (0,)), ((), ())),
                          preferred_element_type=jnp.int32)
    partial = acc.astype(jnp.float32)
    scale = (scale_x * scale_w).astype(jnp.float32)

    return pl.pallas_call(
        _ar_body,
        out_shape=jax.ShapeDtypeStruct((M, N), jnp.float32),
        in_specs=[
            pl.BlockSpec(memory_space=pl.ANY),
            pl.BlockSpec(memory_space=pltpu.MemorySpace.SMEM),
        ],
        out_specs=pl.BlockSpec(memory_space=pl.ANY),
        scratch_shapes=[
            pltpu.VMEM((2, CHUNK, N), jnp.float32),
            pltpu.VMEM((2, CHUNK, N), jnp.float32),
            pltpu.SemaphoreType.DMA((2,)),
            pltpu.SemaphoreType.DMA((2,)),
            pltpu.SemaphoreType.DMA((2,)),
            pltpu.SemaphoreType.DMA,
            pltpu.SemaphoreType.REGULAR,
        ],
        compiler_params=pltpu.CompilerParams(collective_id=0),
    )(partial, scale)


# device time: 1721297 ns/iter; 1.8590x vs baseline; 1.8590x over previous
import jax
import jax.numpy as jnp
from jax import lax
from jax.experimental import pallas as pl
from jax.experimental.pallas import tpu as pltpu

N_DEV = 32
M = 4096
N = 8192
CHUNK = M // N_DEV
HALF = CHUNK // 2
N_STEPS = 2 * (N_DEV - 1)

_PATH_YZ = [(0, 0), (1, 0), (2, 0), (3, 0), (3, 1), (2, 1), (1, 1), (0, 1),
            (0, 2), (1, 2), (2, 2), (3, 2), (3, 3), (2, 3), (1, 3), (0, 3)]
_CYCLE = [(0, y, z) for (y, z) in _PATH_YZ] + \
         [(1, y, z) for (y, z) in reversed(_PATH_YZ)]
_PLANE_ORDER = {(0, 0): 0, (1, 0): 1, (1, 1): 2, (0, 1): 3,
                (0, 2): 4, (1, 2): 5, (1, 3): 6, (0, 3): 7}
_PERM = [8 * z + _PLANE_ORDER[(x, y)] for (x, y, z) in _CYCLE]
_INV = [0] * N_DEV
for _r, _l in enumerate(_PERM):
    _INV[_l] = _r
_RIGHT = [_PERM[(_INV[l] + 1) % N_DEV] for l in range(N_DEV)]
_LEFT = [_PERM[(_INV[l] - 1) % N_DEV] for l in range(N_DEV)]


def _ar_body(partial_ref, scale_ref, meta_ref, out_ref,
             buf_f, buf_r, lbuf_f, lbuf_r,
             send_f, recv_f, send_r, recv_r,
             lsem_f, lsem_r, osem_f, osem_r, credit_f, credit_r):
    r = meta_ref[0]
    rt = meta_ref[1]
    lt = meta_ref[2]

    barrier = pltpu.get_barrier_semaphore()
    for nbr in (lt, rt):
        pl.semaphore_signal(barrier, inc=1, device_id=(nbr,),
                            device_id_type=pl.DeviceIdType.MESH)
    pl.semaphore_wait(barrier, 2)

    cp_f = pltpu.make_async_copy(
        partial_ref.at[pl.ds(r * CHUNK, HALF), :], buf_f.at[0], lsem_f.at[0])
    cp_r = pltpu.make_async_copy(
        partial_ref.at[pl.ds(r * CHUNK + HALF, HALF), :], buf_r.at[0],
        lsem_r.at[0])
    cp_f.start()
    cp_r.start()
    cp_f.wait()
    cp_r.wait()

    scale = scale_ref[0]

    for s in range(N_STEPS):
        a, b = s % 2, (s + 1) % 2
        if s >= 1:
            pl.semaphore_wait(credit_f, 1)
            pl.semaphore_wait(credit_r, 1)
        rdma_f = pltpu.make_async_remote_copy(
            src_ref=buf_f.at[a], dst_ref=buf_f.at[b],
            send_sem=send_f.at[a], recv_sem=recv_f.at[b],
            device_id=(rt,), device_id_type=pl.DeviceIdType.MESH)
        rdma_r = pltpu.make_async_remote_copy(
            src_ref=buf_r.at[a], dst_ref=buf_r.at[b],
            send_sem=send_r.at[a], recv_sem=recv_r.at[b],
            device_id=(lt,), device_id_type=pl.DeviceIdType.MESH)
        rdma_f.start()
        rdma_r.start()

        cf = lax.rem(r - s - 1 + 2 * N_DEV, N_DEV)
        cr = lax.rem(r + s + 1, N_DEV)
        if s < N_DEV - 1:
            lcp_f = pltpu.make_async_copy(
                partial_ref.at[pl.ds(cf * CHUNK, HALF), :],
                lbuf_f.at[b], lsem_f.at[b])
            lcp_r = pltpu.make_async_copy(
                partial_ref.at[pl.ds(cr * CHUNK + HALF, HALF), :],
                lbuf_r.at[b], lsem_r.at[b])
            lcp_f.start()
            lcp_r.start()

        rdma_f.wait_send()
        rdma_r.wait_send()
        if s < N_STEPS - 1:
            pl.semaphore_signal(credit_f, inc=1, device_id=(lt,),
                                device_id_type=pl.DeviceIdType.MESH)
            pl.semaphore_signal(credit_r, inc=1, device_id=(rt,),
                                device_id_type=pl.DeviceIdType.MESH)
        rdma_f.wait_recv()
        rdma_r.wait_recv()

        if s < N_DEV - 2:
            lcp_f.wait()
            buf_f[b] = buf_f[b] + lbuf_f[b]
            lcp_r.wait()
            buf_r[b] = buf_r[b] + lbuf_r[b]
        else:
            if s == N_DEV - 2:
                lcp_f.wait()
                buf_f[b] = jnp.maximum((buf_f[b] + lbuf_f[b]) * scale, 0.0)
                lcp_r.wait()
                buf_r[b] = jnp.maximum((buf_r[b] + lbuf_r[b]) * scale, 0.0)
            ocp_f = pltpu.make_async_copy(
                buf_f.at[b], out_ref.at[pl.ds(cf * CHUNK, HALF), :], osem_f)
            ocp_r = pltpu.make_async_copy(
                buf_r.at[b], out_ref.at[pl.ds(cr * CHUNK + HALF, HALF), :],
                osem_r)
            ocp_f.start()
            ocp_r.start()
            ocp_f.wait()
            ocp_r.wait()


def kernel(x, w_mat, scale_x, scale_w):
    acc = lax.dot_general(x, w_mat, (((1,), (0,)), ((), ())),
                          preferred_element_type=jnp.int32)
    partial = acc.astype(jnp.float32)
    scale = (scale_x * scale_w).astype(jnp.float32)

    p = lax.axis_index("i")
    meta = jnp.stack([
        jnp.asarray(_INV, jnp.int32)[p],
        jnp.asarray(_RIGHT, jnp.int32)[p],
        jnp.asarray(_LEFT, jnp.int32)[p],
    ])

    return pl.pallas_call(
        _ar_body,
        out_shape=jax.ShapeDtypeStruct((M, N), jnp.float32),
        in_specs=[
            pl.BlockSpec(memory_space=pl.ANY),
            pl.BlockSpec(memory_space=pltpu.MemorySpace.SMEM),
            pl.BlockSpec(memory_space=pltpu.MemorySpace.SMEM),
        ],
        out_specs=pl.BlockSpec(memory_space=pl.ANY),
        scratch_shapes=[
            pltpu.VMEM((2, HALF, N), jnp.float32),
            pltpu.VMEM((2, HALF, N), jnp.float32),
            pltpu.VMEM((2, HALF, N), jnp.float32),
            pltpu.VMEM((2, HALF, N), jnp.float32),
            pltpu.SemaphoreType.DMA((2,)),
            pltpu.SemaphoreType.DMA((2,)),
            pltpu.SemaphoreType.DMA((2,)),
            pltpu.SemaphoreType.DMA((2,)),
            pltpu.SemaphoreType.DMA((2,)),
            pltpu.SemaphoreType.DMA((2,)),
            pltpu.SemaphoreType.DMA,
            pltpu.SemaphoreType.DMA,
            pltpu.SemaphoreType.REGULAR,
            pltpu.SemaphoreType.REGULAR,
        ],
        compiler_params=pltpu.CompilerParams(collective_id=0),
    )(partial, scale, meta)


# device time: 1541177 ns/iter; 2.0762x vs baseline; 1.1169x over previous
import jax
import jax.numpy as jnp
from jax import lax
from jax.experimental import pallas as pl
from jax.experimental.pallas import tpu as pltpu

N_DEV = 32
M = 4096
N = 8192
CHUNK = M // N_DEV
HALF = CHUNK // 2
SUB = 2
BAND = HALF // SUB
N_STEPS = 2 * (N_DEV - 1)

_PATH_YZ = [(0, 0), (1, 0), (2, 0), (3, 0), (3, 1), (2, 1), (1, 1), (0, 1),
            (0, 2), (1, 2), (2, 2), (3, 2), (3, 3), (2, 3), (1, 3), (0, 3)]
_CYCLE = [(0, y, z) for (y, z) in _PATH_YZ] + \
         [(1, y, z) for (y, z) in reversed(_PATH_YZ)]
_PLANE_ORDER = {(0, 0): 0, (1, 0): 1, (1, 1): 2, (0, 1): 3,
                (0, 2): 4, (1, 2): 5, (1, 3): 6, (0, 3): 7}
_PERM = [8 * z + _PLANE_ORDER[(x, y)] for (x, y, z) in _CYCLE]
_INV = [0] * N_DEV
for _r, _l in enumerate(_PERM):
    _INV[_l] = _r
_RIGHT = [_PERM[(_INV[l] + 1) % N_DEV] for l in range(N_DEV)]
_LEFT = [_PERM[(_INV[l] - 1) % N_DEV] for l in range(N_DEV)]


def _ar_body(partial_ref, scale_ref, meta_ref, out_ref,
             buf_f, buf_r, lbuf_f, lbuf_r,
             send_f, recv_f, send_r, recv_r,
             lsem_f, lsem_r, osem_f, osem_r, credit_f, credit_r):
    r = meta_ref[0]
    rt = meta_ref[1]
    lt = meta_ref[2]
    scale = scale_ref[0]

    def c_f(s):
        return lax.rem(r - s - 1 + 2 * N_DEV, N_DEV)

    def c_r(s):
        return lax.rem(r + s + 1, N_DEV)

    def start_lcp(s):
        sl = (s + 1) % 2
        f = pltpu.make_async_copy(
            partial_ref.at[pl.ds(c_f(s) * CHUNK, HALF), :],
            lbuf_f.at[sl], lsem_f.at[sl])
        rr = pltpu.make_async_copy(
            partial_ref.at[pl.ds(c_r(s) * CHUNK + HALF, HALF), :],
            lbuf_r.at[sl], lsem_r.at[sl])
        f.start()
        rr.start()
        return f, rr

    def make_rdma(s, k):
        a, b = s % 2, (s + 1) % 2
        k0 = k * BAND
        f = pltpu.make_async_remote_copy(
            src_ref=buf_f.at[a, pl.ds(k0, BAND), :],
            dst_ref=buf_f.at[b, pl.ds(k0, BAND), :],
            send_sem=send_f.at[a, k], recv_sem=recv_f.at[b, k],
            device_id=(rt,), device_id_type=pl.DeviceIdType.MESH)
        rr = pltpu.make_async_remote_copy(
            src_ref=buf_r.at[a, pl.ds(k0, BAND), :],
            dst_ref=buf_r.at[b, pl.ds(k0, BAND), :],
            send_sem=send_r.at[a, k], recv_sem=recv_r.at[b, k],
            device_id=(lt,), device_id_type=pl.DeviceIdType.MESH)
        return f, rr

    init_f = pltpu.make_async_copy(
        partial_ref.at[pl.ds(r * CHUNK, HALF), :], buf_f.at[0], lsem_f.at[0])
    init_r = pltpu.make_async_copy(
        partial_ref.at[pl.ds(r * CHUNK + HALF, HALF), :], buf_r.at[0],
        lsem_r.at[0])
    init_f.start()
    init_r.start()
    lcp = {0: start_lcp(0)}
    init_f.wait()
    init_r.wait()
    lcp[1] = start_lcp(1)

    barrier = pltpu.get_barrier_semaphore()
    for nbr in (lt, rt):
        pl.semaphore_signal(barrier, inc=1, device_id=(nbr,),
                            device_id_type=pl.DeviceIdType.MESH)
    pl.semaphore_wait(barrier, 2)

    pending = {}
    for k in range(SUB):
        pending[k] = make_rdma(0, k)
        pending[k][0].start()
        pending[k][1].start()

    ostores = {}

    for s in range(N_STEPS):
        a, b = s % 2, (s + 1) % 2
        if s <= N_DEV - 2:
            lcp[s][0].wait()
            lcp[s][1].wait()
        nxt = {}
        for k in range(SUB):
            k0 = k * BAND
            rdma_f, rdma_r = pending[k]
            rdma_f.wait_recv()
            rdma_r.wait_recv()
            if s < N_DEV - 2:
                buf_f[b, k0:k0 + BAND, :] = \
                    buf_f[b, k0:k0 + BAND, :] + lbuf_f[b, k0:k0 + BAND, :]
                buf_r[b, k0:k0 + BAND, :] = \
                    buf_r[b, k0:k0 + BAND, :] + lbuf_r[b, k0:k0 + BAND, :]
            elif s == N_DEV - 2:
                buf_f[b, k0:k0 + BAND, :] = jnp.maximum(
                    (buf_f[b, k0:k0 + BAND, :] + lbuf_f[b, k0:k0 + BAND, :])
                    * scale, 0.0)
                buf_r[b, k0:k0 + BAND, :] = jnp.maximum(
                    (buf_r[b, k0:k0 + BAND, :] + lbuf_r[b, k0:k0 + BAND, :])
                    * scale, 0.0)

            rdma_f.wait_send()
            rdma_r.wait_send()
            if s < N_STEPS - 1:
                for ring in ("f", "r"):
                    key = (ring, a, k)
                    if key in ostores:
                        ostores.pop(key).wait()
                pl.semaphore_signal(credit_f, inc=1, device_id=(lt,),
                                    device_id_type=pl.DeviceIdType.MESH)
                pl.semaphore_signal(credit_r, inc=1, device_id=(rt,),
                                    device_id_type=pl.DeviceIdType.MESH)
                pl.semaphore_wait(credit_f, 1)
                pl.semaphore_wait(credit_r, 1)
                nf, nr = make_rdma(s + 1, k)
                nf.start()
                nr.start()
                nxt[k] = (nf, nr)

            if s >= N_DEV - 2:
                of = pltpu.make_async_copy(
                    buf_f.at[b, pl.ds(k0, BAND), :],
                    out_ref.at[pl.ds(c_f(s) * CHUNK + k0, BAND), :],
                    osem_f.at[b, k])
                orr = pltpu.make_async_copy(
                    buf_r.at[b, pl.ds(k0, BAND), :],
                    out_ref.at[pl.ds(c_r(s) * CHUNK + HALF + k0, BAND), :],
                    osem_r.at[b, k])
                of.start()
                orr.start()
                ostores[("f", b, k)] = of
                ostores[("r", b, k)] = orr

        if s <= N_DEV - 4:
            lcp[s + 2] = start_lcp(s + 2)
        pending = nxt

    for od in ostores.values():
        od.wait()


def kernel(x, w_mat, scale_x, scale_w):
    acc = lax.dot_general(x, w_mat, (((1,), (0,)), ((), ())),
                          preferred_element_type=jnp.int32)
    partial = acc.astype(jnp.float32)
    scale = (scale_x * scale_w).astype(jnp.float32)

    p = lax.axis_index("i")
    meta = jnp.stack([
        jnp.asarray(_INV, jnp.int32)[p],
        jnp.asarray(_RIGHT, jnp.int32)[p],
        jnp.asarray(_LEFT, jnp.int32)[p],
    ])

    return pl.pallas_call(
        _ar_body,
        out_shape=jax.ShapeDtypeStruct((M, N), jnp.float32),
        in_specs=[
            pl.BlockSpec(memory_space=pl.ANY),
            pl.BlockSpec(memory_space=pltpu.MemorySpace.SMEM),
            pl.BlockSpec(memory_space=pltpu.MemorySpace.SMEM),
        ],
        out_specs=pl.BlockSpec(memory_space=pl.ANY),
        scratch_shapes=[
            pltpu.VMEM((2, HALF, N), jnp.float32),
            pltpu.VMEM((2, HALF, N), jnp.float32),
            pltpu.VMEM((2, HALF, N), jnp.float32),
            pltpu.VMEM((2, HALF, N), jnp.float32),
            pltpu.SemaphoreType.DMA((2, SUB)),
            pltpu.SemaphoreType.DMA((2, SUB)),
            pltpu.SemaphoreType.DMA((2, SUB)),
            pltpu.SemaphoreType.DMA((2, SUB)),
            pltpu.SemaphoreType.DMA((2,)),
            pltpu.SemaphoreType.DMA((2,)),
            pltpu.SemaphoreType.DMA((2, SUB)),
            pltpu.SemaphoreType.DMA((2, SUB)),
            pltpu.SemaphoreType.REGULAR,
            pltpu.SemaphoreType.REGULAR,
        ],
        compiler_params=pltpu.CompilerParams(collective_id=0),
    )(partial, scale, meta)
